# baseline (speedup 1.0000x reference)
import functools
import jax
import jax.numpy as jnp
from jax import lax
from jax.experimental import pallas as pl
from jax.experimental.pallas import tpu as pltpu
from jax.experimental.pallas import tpu_sc as plsc

_EPS = 1e-05
_B = 16
_C = 96
_H = 224
_W = 224
_HSPLIT = 2
_HB = _H // _HSPLIT
_COUNT = float(_B * _H * _W)

_B_SC = 8
_B_TC = _B - _B_SC
_NW = 32
_FPW = _B_SC * _C // _NW

_mesh = plsc.VectorSubcoreMesh(core_axis_name="c", subcore_axis_name="s")


@functools.partial(
    pl.kernel,
    mesh=_mesh,
    out_type=jax.ShapeDtypeStruct((2, _B_SC, _C, 16), jnp.float32),
    scratch_types=[
        pltpu.VMEM((_H, _W), jnp.float32),
        pltpu.VMEM((_H, _W), jnp.float32),
        pltpu.VMEM((16,), jnp.float32),
        pltpu.SemaphoreType.DMA,
        pltpu.SemaphoreType.DMA,
    ],
)
def _sc_stats(x_hbm, out_hbm, buf0, buf1, tmp_v, sem0, sem1):
    cid = lax.axis_index("c")
    sid = lax.axis_index("s")
    wid = sid * 2 + cid
    base = wid * _FPW
    bufs = (buf0, buf1)
    sems = (sem0, sem1)

    def face_bc(f):
        face = base + f
        return _B_TC + face // _C, face % _C

    def start(f):
        b, c = face_bc(f)
        pltpu.async_copy(x_hbm.at[b, c], bufs[f % 2], sems[f % 2])

    def finish(f):
        b, c = face_bc(f)
        buf = bufs[f % 2]
        pltpu.make_async_copy(x_hbm.at[b, c], buf, sems[f % 2]).wait()

        def acc_loop(r, carry):
            s, s2 = carry
            for k in range(_W // 16):
                t = buf[r, pl.ds(k * 16, 16)]
                s = s + t
                s2 = s2 + t * t
            return s, s2

        z = jnp.zeros((16,), jnp.float32)
        s, s2 = lax.fori_loop(0, _H, acc_loop, (z, z))
        face = base + f
        bb = face // _C
        cc = face % _C
        tmp_v[...] = s
        pltpu.sync_copy(tmp_v, out_hbm.at[0, bb, cc])
        tmp_v[...] = s2
        pltpu.sync_copy(tmp_v, out_hbm.at[1, bb, cc])

    start(0)
    for f in range(_FPW):
        if f + 1 < _FPW:
            start(f + 1)
        finish(f)


def _stats_body(x_ref, out_ref, acc_ref):
    i = pl.program_id(0)
    j = pl.program_id(1)
    first = jnp.logical_and(i == 0, j == 0)
    last = jnp.logical_and(i == _B_TC - 1, j == _HSPLIT - 1)

    @pl.when(first)
    def _():
        acc_ref[...] = jnp.zeros_like(acc_ref)

    x = x_ref[0]
    s = jnp.sum(x, axis=(1, 2))
    s2 = jnp.sum(x * x, axis=(1, 2))
    acc_ref[0, :] += s
    acc_ref[1, :] += s2

    @pl.when(last)
    def _():
        out_ref[...] = acc_ref[...]


def _norm_body(stats_ref, sc_ref, w_ref, b_ref, x_ref, o_ref):
    sc_s = jnp.sum(sc_ref[0], axis=(0, 2))
    sc_q = jnp.sum(sc_ref[1], axis=(0, 2))
    mean = (stats_ref[0, :] + sc_s) / _COUNT
    ex2 = (stats_ref[1, :] + sc_q) / _COUNT
    var = ex2 - mean * mean
    scale = w_ref[0, :] * jax.lax.rsqrt(var + _EPS)
    shift = b_ref[0, :] - mean * scale
    x = x_ref[0]
    o_ref[0] = x * scale[:, None, None] + shift[:, None, None]


def kernel(inputs, weight, bias):
    w = weight.reshape(1, _C)
    b = bias.reshape(1, _C)

    sc_stats = _sc_stats(inputs)

    def x_map(i, j):
        return (i, 0, j, 0)

    tc_stats = pl.pallas_call(
        _stats_body,
        grid=(_B_TC, _HSPLIT),
        in_specs=[pl.BlockSpec((1, _C, _HB, _W), x_map)],
        out_specs=pl.BlockSpec((2, _C), lambda i, j: (0, 0)),
        out_shape=jax.ShapeDtypeStruct((2, _C), jnp.float32),
        scratch_shapes=[pltpu.VMEM((2, _C), jnp.float32)],
        compiler_params=pltpu.CompilerParams(
            dimension_semantics=("arbitrary", "arbitrary"),
        ),
    )(inputs)

    out = pl.pallas_call(
        _norm_body,
        grid=(_B, _HSPLIT),
        in_specs=[
            pl.BlockSpec((2, _C), lambda i, j: (0, 0)),
            pl.BlockSpec((2, _B_SC, _C, 16), lambda i, j: (0, 0, 0, 0)),
            pl.BlockSpec((1, _C), lambda i, j: (0, 0)),
            pl.BlockSpec((1, _C), lambda i, j: (0, 0)),
            pl.BlockSpec((1, _C, _HB, _W), x_map),
        ],
        out_specs=pl.BlockSpec((1, _C, _HB, _W), x_map),
        out_shape=jax.ShapeDtypeStruct((_B, _C, _H, _W), jnp.float32),
        compiler_params=pltpu.CompilerParams(
            dimension_semantics=("arbitrary", "arbitrary"),
        ),
    )(tc_stats, sc_stats, w, b, inputs)

    return out


# speedup vs baseline: 1.0531x; 1.0531x over previous
import functools
import jax
import jax.numpy as jnp
from jax import lax
from jax.experimental import pallas as pl
from jax.experimental.pallas import tpu as pltpu
from jax.experimental.pallas import tpu_sc as plsc

_EPS = 1e-05
_B = 16
_C = 96
_H = 224
_W = 224
_HSPLIT = 2
_HB = _H // _HSPLIT
_COUNT = float(_B * _H * _W)

_B_SC = 2
_B_TC = _B - _B_SC
_NW = 32
_FPW = _B_SC * _C // _NW

_mesh = plsc.VectorSubcoreMesh(core_axis_name="c", subcore_axis_name="s")


@functools.partial(
    pl.kernel,
    mesh=_mesh,
    out_type=jax.ShapeDtypeStruct((2, _B_SC, _C, 16), jnp.float32),
    scratch_types=[
        pltpu.VMEM((_H, _W), jnp.float32),
        pltpu.VMEM((_H, _W), jnp.float32),
        pltpu.VMEM((16,), jnp.float32),
        pltpu.SemaphoreType.DMA,
        pltpu.SemaphoreType.DMA,
    ],
)
def _sc_stats(x_hbm, out_hbm, buf0, buf1, tmp_v, sem0, sem1):
    cid = lax.axis_index("c")
    sid = lax.axis_index("s")
    wid = sid * 2 + cid
    base = wid * _FPW
    bufs = (buf0, buf1)
    sems = (sem0, sem1)

    def face_bc(f):
        face = base + f
        return _B_TC + face // _C, face % _C

    def start(f):
        b, c = face_bc(f)
        pltpu.async_copy(x_hbm.at[b, c], bufs[f % 2], sems[f % 2])

    def finish(f):
        b, c = face_bc(f)
        buf = bufs[f % 2]
        pltpu.make_async_copy(x_hbm.at[b, c], buf, sems[f % 2]).wait()

        def acc_loop(r, carry):
            s, s2 = carry
            for k in range(_W // 16):
                t = buf[r, pl.ds(k * 16, 16)]
                s = s + t
                s2 = s2 + t * t
            return s, s2

        z = jnp.zeros((16,), jnp.float32)
        s, s2 = lax.fori_loop(0, _H, acc_loop, (z, z))
        face = base + f
        bb = face // _C
        cc = face % _C
        tmp_v[...] = s
        pltpu.sync_copy(tmp_v, out_hbm.at[0, bb, cc])
        tmp_v[...] = s2
        pltpu.sync_copy(tmp_v, out_hbm.at[1, bb, cc])

    start(0)
    for f in range(_FPW):
        if f + 1 < _FPW:
            start(f + 1)
        finish(f)


def _stats_body(x_ref, out_ref, acc_ref):
    i = pl.program_id(0)
    j = pl.program_id(1)
    first = jnp.logical_and(i == 0, j == 0)
    last = jnp.logical_and(i == _B_TC - 1, j == _HSPLIT - 1)

    @pl.when(first)
    def _():
        acc_ref[...] = jnp.zeros_like(acc_ref)

    x = x_ref[0]
    s = jnp.sum(x, axis=(1, 2))
    s2 = jnp.sum(x * x, axis=(1, 2))
    acc_ref[0, :] += s
    acc_ref[1, :] += s2

    @pl.when(last)
    def _():
        out_ref[...] = acc_ref[...]


def _norm_body(stats_ref, sc_ref, w_ref, b_ref, x_ref, o_ref):
    sc_s = jnp.sum(sc_ref[0], axis=(0, 2))
    sc_q = jnp.sum(sc_ref[1], axis=(0, 2))
    mean = (stats_ref[0, :] + sc_s) / _COUNT
    ex2 = (stats_ref[1, :] + sc_q) / _COUNT
    var = ex2 - mean * mean
    scale = w_ref[0, :] * jax.lax.rsqrt(var + _EPS)
    shift = b_ref[0, :] - mean * scale
    x = x_ref[0]
    o_ref[0] = x * scale[:, None, None] + shift[:, None, None]


def kernel(inputs, weight, bias):
    w = weight.reshape(1, _C)
    b = bias.reshape(1, _C)

    sc_stats = _sc_stats(inputs)

    def x_map(i, j):
        return (i, 0, j, 0)

    tc_stats = pl.pallas_call(
        _stats_body,
        grid=(_B_TC, _HSPLIT),
        in_specs=[pl.BlockSpec((1, _C, _HB, _W), x_map)],
        out_specs=pl.BlockSpec((2, _C), lambda i, j: (0, 0)),
        out_shape=jax.ShapeDtypeStruct((2, _C), jnp.float32),
        scratch_shapes=[pltpu.VMEM((2, _C), jnp.float32)],
        compiler_params=pltpu.CompilerParams(
            dimension_semantics=("arbitrary", "arbitrary"),
        ),
    )(inputs)

    out = pl.pallas_call(
        _norm_body,
        grid=(_B, _HSPLIT),
        in_specs=[
            pl.BlockSpec((2, _C), lambda i, j: (0, 0)),
            pl.BlockSpec((2, _B_SC, _C, 16), lambda i, j: (0, 0, 0, 0)),
            pl.BlockSpec((1, _C), lambda i, j: (0, 0)),
            pl.BlockSpec((1, _C), lambda i, j: (0, 0)),
            pl.BlockSpec((1, _C, _HB, _W), x_map),
        ],
        out_specs=pl.BlockSpec((1, _C, _HB, _W), x_map),
        out_shape=jax.ShapeDtypeStruct((_B, _C, _H, _W), jnp.float32),
        compiler_params=pltpu.CompilerParams(
            dimension_semantics=("arbitrary", "arbitrary"),
        ),
    )(tc_stats, sc_stats, w, b, inputs)

    return out


# speedup vs baseline: 1.0589x; 1.0055x over previous
import functools
import jax
import jax.numpy as jnp
from jax import lax
from jax.experimental import pallas as pl
from jax.experimental.pallas import tpu as pltpu
from jax.experimental.pallas import tpu_sc as plsc

_EPS = 1e-05
_B = 16
_C = 96
_H = 224
_W = 224
_HSPLIT = 2
_HB = _H // _HSPLIT
_COUNT = float(_B * _H * _W)

_B_SC = 1
_B_TC = _B - _B_SC
_NW = 32
_FPW = _B_SC * _C // _NW

_mesh = plsc.VectorSubcoreMesh(core_axis_name="c", subcore_axis_name="s")


@functools.partial(
    pl.kernel,
    mesh=_mesh,
    out_type=jax.ShapeDtypeStruct((2, _B_SC, _C, 16), jnp.float32),
    scratch_types=[
        pltpu.VMEM((_H, _W), jnp.float32),
        pltpu.VMEM((_H, _W), jnp.float32),
        pltpu.VMEM((16,), jnp.float32),
        pltpu.SemaphoreType.DMA,
        pltpu.SemaphoreType.DMA,
    ],
)
def _sc_stats(x_hbm, out_hbm, buf0, buf1, tmp_v, sem0, sem1):
    cid = lax.axis_index("c")
    sid = lax.axis_index("s")
    wid = sid * 2 + cid
    base = wid * _FPW
    bufs = (buf0, buf1)
    sems = (sem0, sem1)

    def face_bc(f):
        face = base + f
        return _B_TC + face // _C, face % _C

    def start(f):
        b, c = face_bc(f)
        pltpu.async_copy(x_hbm.at[b, c], bufs[f % 2], sems[f % 2])

    def finish(f):
        b, c = face_bc(f)
        buf = bufs[f % 2]
        pltpu.make_async_copy(x_hbm.at[b, c], buf, sems[f % 2]).wait()

        def acc_loop(r, carry):
            s, s2 = carry
            for k in range(_W // 16):
                t = buf[r, pl.ds(k * 16, 16)]
                s = s + t
                s2 = s2 + t * t
            return s, s2

        z = jnp.zeros((16,), jnp.float32)
        s, s2 = lax.fori_loop(0, _H, acc_loop, (z, z))
        face = base + f
        bb = face // _C
        cc = face % _C
        tmp_v[...] = s
        pltpu.sync_copy(tmp_v, out_hbm.at[0, bb, cc])
        tmp_v[...] = s2
        pltpu.sync_copy(tmp_v, out_hbm.at[1, bb, cc])

    start(0)
    for f in range(_FPW):
        if f + 1 < _FPW:
            start(f + 1)
        finish(f)


def _stats_body(x_ref, out_ref, acc_ref):
    i = pl.program_id(0)
    j = pl.program_id(1)
    first = jnp.logical_and(i == 0, j == 0)
    last = jnp.logical_and(i == _B_TC - 1, j == _HSPLIT - 1)

    @pl.when(first)
    def _():
        acc_ref[...] = jnp.zeros_like(acc_ref)

    x = x_ref[0]
    s = jnp.sum(x, axis=(1, 2))
    s2 = jnp.sum(x * x, axis=(1, 2))
    acc_ref[0, :] += s
    acc_ref[1, :] += s2

    @pl.when(last)
    def _():
        out_ref[...] = acc_ref[...]


def _norm_body(stats_ref, sc_ref, w_ref, b_ref, x_ref, o_ref):
    sc_s = jnp.sum(sc_ref[0], axis=(0, 2))
    sc_q = jnp.sum(sc_ref[1], axis=(0, 2))
    mean = (stats_ref[0, :] + sc_s) / _COUNT
    ex2 = (stats_ref[1, :] + sc_q) / _COUNT
    var = ex2 - mean * mean
    scale = w_ref[0, :] * jax.lax.rsqrt(var + _EPS)
    shift = b_ref[0, :] - mean * scale
    x = x_ref[0]
    o_ref[0] = x * scale[:, None, None] + shift[:, None, None]


def kernel(inputs, weight, bias):
    w = weight.reshape(1, _C)
    b = bias.reshape(1, _C)

    sc_stats = _sc_stats(inputs)

    def x_map(i, j):
        return (i, 0, j, 0)

    tc_stats = pl.pallas_call(
        _stats_body,
        grid=(_B_TC, _HSPLIT),
        in_specs=[pl.BlockSpec((1, _C, _HB, _W), x_map)],
        out_specs=pl.BlockSpec((2, _C), lambda i, j: (0, 0)),
        out_shape=jax.ShapeDtypeStruct((2, _C), jnp.float32),
        scratch_shapes=[pltpu.VMEM((2, _C), jnp.float32)],
        compiler_params=pltpu.CompilerParams(
            dimension_semantics=("arbitrary", "arbitrary"),
        ),
    )(inputs)

    out = pl.pallas_call(
        _norm_body,
        grid=(_B, _HSPLIT),
        in_specs=[
            pl.BlockSpec((2, _C), lambda i, j: (0, 0)),
            pl.BlockSpec((2, _B_SC, _C, 16), lambda i, j: (0, 0, 0, 0)),
            pl.BlockSpec((1, _C), lambda i, j: (0, 0)),
            pl.BlockSpec((1, _C), lambda i, j: (0, 0)),
            pl.BlockSpec((1, _C, _HB, _W), x_map),
        ],
        out_specs=pl.BlockSpec((1, _C, _HB, _W), x_map),
        out_shape=jax.ShapeDtypeStruct((_B, _C, _H, _W), jnp.float32),
        compiler_params=pltpu.CompilerParams(
            dimension_semantics=("arbitrary", "arbitrary"),
        ),
    )(tc_stats, sc_stats, w, b, inputs)

    return out
